# Initial kernel scaffold; baseline (speedup 1.0000x reference)
import jax
import jax.numpy as jnp
from jax.experimental import pallas as pl


def kernel(codes, codebooks, scales):
    raise NotImplementedError("write your pallas kernel here")


# speedup vs baseline: 34.3846x; 34.3846x over previous
import functools

import jax
import jax.numpy as jnp
from jax import lax
from jax.experimental import pallas as pl
from jax.experimental.pallas import tpu as pltpu
from jax.experimental.pallas import tpu_sc as plsc

_NC, _NS, _L = 2, 16, 16
_NW = _NC * _NS
_BLK = 128
_RB = 4


@functools.lru_cache(maxsize=None)
def _make_sc_dequant(num_rows, codes_per_row, d):
    rows_w = num_rows // _NW
    codes_w = rows_w * codes_per_row
    chunk = _RB * codes_per_row
    nblk = chunk // _BLK
    nb = codes_w // chunk
    npair = nb // 2
    blocks_total = (num_rows * codes_per_row) // _BLK
    blk_per_row = codes_per_row // _BLK

    mesh = plsc.VectorSubcoreMesh(
        core_axis_name="c", subcore_axis_name="s",
        num_cores=_NC, num_subcores=_NS)

    def body(codes_hbm, table_hbm, scales_hbm, out_hbm,
             idx0, idx1, g0, g1, scales_v, gsem, osem0, osem1):
        w = lax.axis_index("s") * _NC + lax.axis_index("c")
        row0 = w * rows_w
        blk0 = w * (codes_w // _BLK)

        pltpu.sync_copy(scales_hbm.at[pl.ds(row0, rows_w)], scales_v)

        def load_codes(b, idx_v):
            boff = blk0 + b * nblk
            pltpu.sync_copy(codes_hbm.at[pl.ds(boff, nblk)], idx_v)

        lanes = lax.iota(jnp.int32, _L)
        kbase = lax.shift_right_logical(lanes, 3)
        dvec = lax.bitwise_and(lanes, 7)

        def scale_block(gref, svec):
            @plsc.parallel_loop(0, _BLK // 2, unroll=8)
            def _(g):
                kvec = kbase + 2 * g
                v = plsc.load_gather(gref, [kvec, dvec])
                plsc.store_scatter(gref, [kvec, dvec], v * svec)

        def do_batch(b, idx_v, gbuf, osem):
            hs = [pltpu.async_copy(table_hbm.at[idx_v.at[j]], gbuf.at[j], gsem)
                  for j in range(nblk)]
            for j in range(nblk):
                hs[j].wait()
            for j in range(nblk):
                lrow = b * _RB + (j // blk_per_row)
                svec = scales_v[lrow]
                scale_block(gbuf.at[j], svec)
            boff = blk0 + b * nblk
            pltpu.async_copy(gbuf, out_hbm.at[pl.ds(boff, nblk)], osem)

        def drain_out(b, gbuf, osem):
            boff = blk0 + b * nblk
            pltpu.make_async_copy(gbuf, out_hbm.at[pl.ds(boff, nblk)], osem).wait()

        def pair(p, carry):
            b0 = 2 * p
            b1 = b0 + 1

            @pl.when(p > 0)
            def _():
                drain_out(b0 - 2, g0, osem0)
            load_codes(b0, idx0)
            do_batch(b0, idx0, g0, osem0)

            @pl.when(p > 0)
            def _():
                drain_out(b1 - 2, g1, osem1)
            load_codes(b1, idx1)
            do_batch(b1, idx1, g1, osem1)
            return carry

        lax.fori_loop(0, npair, pair, jnp.int32(0))
        drain_out(nb - 2, g0, osem0)
        drain_out(nb - 1, g1, osem1)

    scratch = [
        pltpu.VMEM((nblk, _BLK), jnp.int32),
        pltpu.VMEM((nblk, _BLK), jnp.int32),
        pltpu.VMEM((nblk, _BLK, d), jnp.float32),
        pltpu.VMEM((nblk, _BLK, d), jnp.float32),
        pltpu.VMEM((rows_w, _L), jnp.float32),
        pltpu.SemaphoreType.DMA,
        pltpu.SemaphoreType.DMA,
        pltpu.SemaphoreType.DMA,
    ]
    out_type = jax.ShapeDtypeStruct((blocks_total, _BLK, d), jnp.float32)
    return pl.kernel(body, out_type=out_type, mesh=mesh, scratch_types=scratch,
                     compiler_params=pltpu.CompilerParams(
                         use_tc_tiling_on_sc=False,
                         needs_layout_passes=False))


def kernel(codes, codebooks, scales):
    num_out_groups, num_in_groups, num_codebooks = codes.shape
    _, codebook_size, out_group_size, in_group_size = codebooks.shape
    d = out_group_size * in_group_size
    codes2 = codes.reshape(num_out_groups * num_in_groups // _BLK, _BLK)
    table = codebooks.reshape(num_codebooks * codebook_size, d)
    scal = jnp.broadcast_to(scales.reshape(num_out_groups, 1),
                            (num_out_groups, _L))
    fn = _make_sc_dequant(num_out_groups, num_in_groups, d)
    out = fn(codes2, table, scal)
    return out.reshape(num_out_groups, num_in_groups * d)


# speedup vs baseline: 39.3067x; 1.1431x over previous
import functools

import jax
import jax.numpy as jnp
from jax import lax
from jax.experimental import pallas as pl
from jax.experimental.pallas import tpu as pltpu
from jax.experimental.pallas import tpu_sc as plsc

_NC, _NS, _L = 2, 16, 16
_NW = _NC * _NS
_BLK = 128
_RB = 8


@functools.lru_cache(maxsize=None)
def _make_sc_dequant(num_rows, codes_per_row, d):
    rows_w = num_rows // _NW
    codes_w = rows_w * codes_per_row
    chunk = _RB * codes_per_row
    nblk = chunk // _BLK
    half = nblk // 2
    nb = codes_w // chunk
    npair = nb // 2
    blocks_total = (num_rows * codes_per_row) // _BLK
    blk_per_row = codes_per_row // _BLK

    mesh = plsc.VectorSubcoreMesh(
        core_axis_name="c", subcore_axis_name="s",
        num_cores=_NC, num_subcores=_NS)

    def body(codes_hbm, table_hbm, scales_hbm, out_hbm,
             idx0, idx1, g0, g1, scales_v,
             gsem_a, gsem_b, osem0, osem1, isem):
        w = lax.axis_index("s") * _NC + lax.axis_index("c")
        row0 = w * rows_w
        blk0 = w * (codes_w // _BLK)

        pltpu.sync_copy(scales_hbm.at[pl.ds(row0, rows_w)], scales_v)

        lanes = lax.iota(jnp.int32, _L)
        kbase = lax.shift_right_logical(lanes, 3)
        dvec = lax.bitwise_and(lanes, 7)

        def idx_copy(b, idx_v):
            boff = blk0 + b * nblk
            return pltpu.make_async_copy(
                codes_hbm.at[pl.ds(boff, nblk)], idx_v, isem)

        def scale_block(gref, svec):
            @plsc.parallel_loop(0, _BLK // 2, unroll=8)
            def _(g):
                kvec = kbase + 2 * g
                v = plsc.load_gather(gref, [kvec, dvec])
                plsc.store_scatter(gref, [kvec, dvec], v * svec)

        def scale_half(b, gbuf, j0):
            for j in range(j0, j0 + half):
                svec = scales_v[b * _RB + (j // blk_per_row)]
                scale_block(gbuf.at[j], svec)

        def do_batch(b, idx_v, gbuf, osem, prefetch):
            hs_a = [pltpu.async_copy(table_hbm.at[idx_v.at[j]], gbuf.at[j],
                                     gsem_a)
                    for j in range(half)]
            hs_b = [pltpu.async_copy(table_hbm.at[idx_v.at[j]], gbuf.at[j],
                                     gsem_b)
                    for j in range(half, nblk)]
            if prefetch is not None:
                prefetch()
            for h in hs_a:
                h.wait()
            scale_half(b, gbuf, 0)
            for h in hs_b:
                h.wait()
            scale_half(b, gbuf, half)
            boff = blk0 + b * nblk
            pltpu.async_copy(gbuf, out_hbm.at[pl.ds(boff, nblk)], osem)

        def drain_out(b, gbuf, osem):
            boff = blk0 + b * nblk
            pltpu.make_async_copy(gbuf, out_hbm.at[pl.ds(boff, nblk)],
                                  osem).wait()

        def pair(p, carry):
            b0 = 2 * p
            b1 = b0 + 1

            @pl.when(p > 0)
            def _():
                drain_out(b0 - 2, g0, osem0)

            def prefetch_idx1():
                idx_copy(b1, idx1).start()
            do_batch(b0, idx0, g0, osem0, prefetch_idx1)
            idx_copy(b1, idx1).wait()

            @pl.when(p > 0)
            def _():
                drain_out(b1 - 2, g1, osem1)

            @pl.when(p + 1 < npair)
            def _():
                idx_copy(b0 + 2, idx0).start()
            do_batch(b1, idx1, g1, osem1, None)

            @pl.when(p + 1 < npair)
            def _():
                idx_copy(b0 + 2, idx0).wait()
            return carry

        idx_copy(0, idx0).start()
        idx_copy(0, idx0).wait()
        lax.fori_loop(0, npair, pair, jnp.int32(0))
        drain_out(nb - 2, g0, osem0)
        drain_out(nb - 1, g1, osem1)

    scratch = [
        pltpu.VMEM((nblk, _BLK), jnp.int32),
        pltpu.VMEM((nblk, _BLK), jnp.int32),
        pltpu.VMEM((nblk, _BLK, d), jnp.float32),
        pltpu.VMEM((nblk, _BLK, d), jnp.float32),
        pltpu.VMEM((rows_w, _L), jnp.float32),
        pltpu.SemaphoreType.DMA,
        pltpu.SemaphoreType.DMA,
        pltpu.SemaphoreType.DMA,
        pltpu.SemaphoreType.DMA,
        pltpu.SemaphoreType.DMA,
    ]
    out_type = jax.ShapeDtypeStruct((blocks_total, _BLK, d), jnp.float32)
    return pl.kernel(body, out_type=out_type, mesh=mesh, scratch_types=scratch,
                     compiler_params=pltpu.CompilerParams(
                         use_tc_tiling_on_sc=False,
                         needs_layout_passes=False))


def kernel(codes, codebooks, scales):
    num_out_groups, num_in_groups, num_codebooks = codes.shape
    _, codebook_size, out_group_size, in_group_size = codebooks.shape
    d = out_group_size * in_group_size
    codes2 = codes.reshape(num_out_groups * num_in_groups // _BLK, _BLK)
    table = codebooks.reshape(num_codebooks * codebook_size, d)
    scal = jnp.broadcast_to(scales.reshape(num_out_groups, 1),
                            (num_out_groups, _L))
    fn = _make_sc_dequant(num_out_groups, num_in_groups, d)
    out = fn(codes2, table, scal)
    return out.reshape(num_out_groups, num_in_groups * d)


# speedup vs baseline: 50.0973x; 1.2745x over previous
import functools

import jax
import jax.numpy as jnp
from jax import lax
from jax.experimental import pallas as pl
from jax.experimental.pallas import tpu as pltpu
from jax.experimental.pallas import tpu_sc as plsc

_NC, _NS, _L = 2, 16, 16
_NW = _NC * _NS
_BLK = 128
_RB = 8


@functools.lru_cache(maxsize=None)
def _make_sc_dequant(num_rows, codes_per_row, d):
    rows_w = num_rows // _NW
    codes_w = rows_w * codes_per_row
    chunk = _RB * codes_per_row
    nblk = chunk // _BLK
    half = nblk // 2
    nb = codes_w // chunk
    npair = nb // 2
    blocks_total = (num_rows * codes_per_row) // _BLK
    blk_per_row = codes_per_row // _BLK

    mesh = plsc.VectorSubcoreMesh(
        core_axis_name="c", subcore_axis_name="s",
        num_cores=_NC, num_subcores=_NS)

    def body(codes_hbm, table_hbm, scales_hbm, out_hbm,
             idx0, idx1, g0, g1, scales_v, table_sh,
             gsem_a, gsem_b, osem0, osem1, isem):
        sid = lax.axis_index("s")
        w = sid * _NC + lax.axis_index("c")
        row0 = w * rows_w
        blk0 = w * (codes_w // _BLK)

        tslc = 65536 // _NS
        pltpu.sync_copy(table_hbm.at[pl.ds(sid * tslc, tslc)],
                        table_sh.at[pl.ds(sid * tslc, tslc)])
        plsc.subcore_barrier()

        pltpu.sync_copy(scales_hbm.at[pl.ds(row0, rows_w)], scales_v)

        lanes = lax.iota(jnp.int32, _L)
        kbase = lax.shift_right_logical(lanes, 3)
        dvec = lax.bitwise_and(lanes, 7)

        def idx_copy(b, idx_v):
            boff = blk0 + b * nblk
            return pltpu.make_async_copy(
                codes_hbm.at[pl.ds(boff, nblk)], idx_v, isem)

        def scale_block(gref, svec):
            @plsc.parallel_loop(0, _BLK // 2, unroll=8)
            def _(g):
                kvec = kbase + 2 * g
                v = plsc.load_gather(gref, [kvec, dvec])
                plsc.store_scatter(gref, [kvec, dvec], v * svec)

        def scale_half(b, gbuf, j0):
            for j in range(j0, j0 + half):
                svec = scales_v[b * _RB + (j // blk_per_row)]
                scale_block(gbuf.at[j], svec)

        def do_batch(b, idx_v, gbuf, osem, prefetch):
            hs_a = [pltpu.async_copy(table_sh.at[idx_v.at[j]], gbuf.at[j],
                                     gsem_a)
                    for j in range(half)]
            hs_b = [pltpu.async_copy(table_sh.at[idx_v.at[j]], gbuf.at[j],
                                     gsem_b)
                    for j in range(half, nblk)]
            if prefetch is not None:
                prefetch()
            for h in hs_a:
                h.wait()
            scale_half(b, gbuf, 0)
            for h in hs_b:
                h.wait()
            scale_half(b, gbuf, half)
            boff = blk0 + b * nblk
            pltpu.async_copy(gbuf, out_hbm.at[pl.ds(boff, nblk)], osem)

        def drain_out(b, gbuf, osem):
            boff = blk0 + b * nblk
            pltpu.make_async_copy(gbuf, out_hbm.at[pl.ds(boff, nblk)],
                                  osem).wait()

        def pair(p, carry):
            b0 = 2 * p
            b1 = b0 + 1

            @pl.when(p > 0)
            def _():
                drain_out(b0 - 2, g0, osem0)

            def prefetch_idx1():
                idx_copy(b1, idx1).start()
            do_batch(b0, idx0, g0, osem0, prefetch_idx1)
            idx_copy(b1, idx1).wait()

            @pl.when(p > 0)
            def _():
                drain_out(b1 - 2, g1, osem1)

            @pl.when(p + 1 < npair)
            def _():
                idx_copy(b0 + 2, idx0).start()
            do_batch(b1, idx1, g1, osem1, None)

            @pl.when(p + 1 < npair)
            def _():
                idx_copy(b0 + 2, idx0).wait()
            return carry

        idx_copy(0, idx0).start()
        idx_copy(0, idx0).wait()
        lax.fori_loop(0, npair, pair, jnp.int32(0))
        drain_out(nb - 2, g0, osem0)
        drain_out(nb - 1, g1, osem1)

    scratch = [
        pltpu.VMEM((nblk, _BLK), jnp.int32),
        pltpu.VMEM((nblk, _BLK), jnp.int32),
        pltpu.VMEM((nblk, _BLK, d), jnp.float32),
        pltpu.VMEM((nblk, _BLK, d), jnp.float32),
        pltpu.VMEM((rows_w, _L), jnp.float32),
        pltpu.VMEM_SHARED((65536, d), jnp.float32),
        pltpu.SemaphoreType.DMA,
        pltpu.SemaphoreType.DMA,
        pltpu.SemaphoreType.DMA,
        pltpu.SemaphoreType.DMA,
        pltpu.SemaphoreType.DMA,
    ]
    out_type = jax.ShapeDtypeStruct((blocks_total, _BLK, d), jnp.float32)
    return pl.kernel(body, out_type=out_type, mesh=mesh, scratch_types=scratch,
                     compiler_params=pltpu.CompilerParams(
                         use_tc_tiling_on_sc=False,
                         needs_layout_passes=False))


def kernel(codes, codebooks, scales):
    num_out_groups, num_in_groups, num_codebooks = codes.shape
    _, codebook_size, out_group_size, in_group_size = codebooks.shape
    d = out_group_size * in_group_size
    codes2 = codes.reshape(num_out_groups * num_in_groups // _BLK, _BLK)
    table = codebooks.reshape(num_codebooks * codebook_size, d)
    scal = jnp.broadcast_to(scales.reshape(num_out_groups, 1),
                            (num_out_groups, _L))
    fn = _make_sc_dequant(num_out_groups, num_in_groups, d)
    out = fn(codes2, table, scal)
    return out.reshape(num_out_groups, num_in_groups * d)
